# baseline (speedup 1.0000x reference)
import functools

import jax
import jax.numpy as jnp
from jax import lax
from jax.experimental import pallas as pl
from jax.experimental.pallas import tpu as pltpu
from jax.experimental.pallas import tpu_sc as plsc

N = 50000
E = 800000
F_IN = 8
OUT = 32
PERIODS = 12
FW = F_IN * PERIODS

NCORE = 2
NSUB = 16
NPAD = 50176
RPT = NPAD // NSUB

GROUP = 128
G0 = 96
G1 = 312
NGRP = NSUB * (G0 + G1)
EPAD = NGRP * GROUP

NCHUNK = 3
CW = 32

BLK = 512
NBLK = NPAD // BLK

_mesh = plsc.VectorSubcoreMesh(core_axis_name="c", subcore_axis_name="s")


DW = 16
DGA = 208
DGB = NGRP // NSUB - DGA


@functools.partial(
    pl.kernel,
    out_type=jax.ShapeDtypeStruct((NCORE, NPAD, DW), jnp.float32),
    mesh=_mesh,
    compiler_params=pltpu.CompilerParams(use_tc_tiling_on_sc=False),
    scratch_types=[
        pltpu.VMEM((DGA, GROUP), jnp.int32),
        pltpu.VMEM((GROUP, DW), jnp.float32),
        pltpu.VMEM_SHARED((NPAD, DW), jnp.float32),
    ],
)
def _deg_kernel(dst_hbm, ones_hbm, zeros1_hbm, deg_out, didx, ones_v, deg_sh):
    c = lax.axis_index("c")
    s = lax.axis_index("s")
    rbase = pl.multiple_of(s * RPT, 8)
    pltpu.sync_copy(zeros1_hbm.at[pl.ds(rbase, RPT)], deg_sh.at[pl.ds(rbase, RPT)])
    pltpu.sync_copy(ones_hbm, ones_v)

    def run_core(ngr, gbase):
        pltpu.sync_copy(dst_hbm.at[pl.ds(gbase, ngr)], didx.at[pl.ds(0, ngr)])
        plsc.subcore_barrier()

        @pl.loop(0, ngr)
        def _scatter(g):
            pltpu.sync_copy(ones_v, deg_sh.at[didx.at[g]], add=True)

    @pl.when(c == 0)
    def _():
        run_core(DGA, pl.multiple_of(s * DGA, 8))

    @pl.when(c == 1)
    def _():
        run_core(DGB, pl.multiple_of(NSUB * DGA + s * DGB, 8))

    plsc.subcore_barrier()
    pltpu.sync_copy(deg_sh.at[pl.ds(rbase, RPT)], deg_out.at[c, pl.ds(rbase, RPT)])


def _prep_body(deg_ref, x_ref, dinv_ref, xs0_ref, xs1_ref, xs2_ref):
    d = deg_ref[0, :, 0:1] + deg_ref[1, :, 0:1] + 1.0
    dv = lax.rsqrt(d)
    dinv_ref[...] = dv
    x96 = x_ref[...].reshape(x_ref.shape[0], FW)
    xs0_ref[...] = dv * x96[:, 0 * CW:1 * CW]
    xs1_ref[...] = dv * x96[:, 1 * CW:2 * CW]
    xs2_ref[...] = dv * x96[:, 2 * CW:3 * CW]


_prep = pl.pallas_call(
    _prep_body,
    grid=(NBLK,),
    in_specs=[
        pl.BlockSpec((NCORE, BLK, DW), lambda i: (0, i, 0)),
        pl.BlockSpec((BLK, F_IN, PERIODS), lambda i: (i, 0, 0)),
    ],
    out_specs=[
        pl.BlockSpec((BLK, 1), lambda i: (i, 0)),
        pl.BlockSpec((BLK, CW), lambda i: (i, 0)),
        pl.BlockSpec((BLK, CW), lambda i: (i, 0)),
        pl.BlockSpec((BLK, CW), lambda i: (i, 0)),
    ],
    out_shape=[
        jax.ShapeDtypeStruct((NPAD, 1), jnp.float32),
        jax.ShapeDtypeStruct((NPAD, CW), jnp.float32),
        jax.ShapeDtypeStruct((NPAD, CW), jnp.float32),
        jax.ShapeDtypeStruct((NPAD, CW), jnp.float32),
    ],
)


IB = 24
NIB0 = G0 // IB
NIB1 = G1 // IB


@functools.partial(
    pl.kernel,
    out_type=[jax.ShapeDtypeStruct((NPAD, CW), jnp.float32)] * (NCORE * NCHUNK),
    mesh=_mesh,
    compiler_params=pltpu.CompilerParams(use_tc_tiling_on_sc=False),
    scratch_types=[
        pltpu.VMEM((IB, GROUP), jnp.int32),
        pltpu.VMEM((IB, GROUP), jnp.int32),
        pltpu.VMEM((GROUP, CW), jnp.float32),
        pltpu.VMEM((GROUP, CW), jnp.float32),
        pltpu.VMEM((GROUP, CW), jnp.float32),
        pltpu.VMEM((GROUP, CW), jnp.float32),
        pltpu.VMEM_SHARED((NPAD, CW), jnp.float32),
        pltpu.SemaphoreType.DMA,
        pltpu.SemaphoreType.DMA,
        pltpu.SemaphoreType.DMA,
        pltpu.SemaphoreType.DMA,
    ],
)
def _msg_kernel(src_hbm, dst_hbm, xs0_hbm, xs1_hbm, xs2_hbm, zeros_hbm,
                y00, y01, y02, y10, y11, y12,
                sidx, didx, rows0, rows1, rows2, rows3, y_sh,
                sem0, sem1, sem2, sem3):
    c = lax.axis_index("c")
    s = lax.axis_index("s")
    rows = (rows0, rows1, rows2, rows3)
    sems = (sem0, sem1, sem2, sem3)
    rbase = pl.multiple_of(s * RPT, 8)

    def run_core(nib, gbase, outs):
        for chunk, xs_hbm in enumerate((xs0_hbm, xs1_hbm, xs2_hbm)):
            pltpu.sync_copy(zeros_hbm, y_sh.at[pl.ds(rbase, RPT)])
            plsc.subcore_barrier()

            @pl.loop(0, nib)
            def _blocks(blk):
                bbase = pl.multiple_of(gbase + blk * IB, 8)
                pltpu.sync_copy(src_hbm.at[pl.ds(bbase, IB)], sidx)
                pltpu.sync_copy(dst_hbm.at[pl.ds(bbase, IB)], didx)

                for b in range(4):
                    pltpu.async_copy(xs_hbm.at[sidx.at[b]], rows[b], sems[b])

                @pl.loop(0, IB - 4, step=4)
                def _groups(g0):
                    for b in range(4):
                        g = g0 + b
                        pltpu.make_async_copy(xs_hbm.at[sidx.at[g]], rows[b],
                                              sems[b]).wait()
                        pltpu.sync_copy(rows[b], y_sh.at[didx.at[g]], add=True)
                        pltpu.async_copy(xs_hbm.at[sidx.at[g + 4]], rows[b],
                                         sems[b])

                for b in range(4):
                    g = IB - 4 + b
                    pltpu.make_async_copy(xs_hbm.at[sidx.at[g]], rows[b],
                                          sems[b]).wait()
                    pltpu.sync_copy(rows[b], y_sh.at[didx.at[g]], add=True)

            plsc.subcore_barrier()
            pltpu.sync_copy(y_sh.at[pl.ds(rbase, RPT)],
                            outs[chunk].at[pl.ds(rbase, RPT)])

    @pl.when(c == 0)
    def _():
        run_core(NIB0, pl.multiple_of(s * G0, 8), (y00, y01, y02))

    @pl.when(c == 1)
    def _():
        run_core(NIB1, pl.multiple_of(NSUB * G0 + s * G1, 8), (y10, y11, y12))


def _dense_body(dinv_ref, y00, y01, y02, y10, y11, y12,
                xs0_ref, xs1_ref, xs2_ref,
                azb_ref, ahb_ref, czb_ref, chb_ref, wsum_ref, wo_ref, bo_ref,
                out_ref):
    dv = dinv_ref[...]
    xs = (xs0_ref, xs1_ref, xs2_ref)
    yp0 = (y00, y01, y02)
    yp1 = (y10, y11, y12)
    y = [dv * (yp0[cc][...] + yp1[cc][...] + xs[cc][...]) for cc in range(NCHUNK)]
    Y = jnp.concatenate(y, axis=1)
    Pz = jnp.dot(Y, azb_ref[...], preferred_element_type=jnp.float32) + czb_ref[...]
    Ph = jnp.dot(Y, ahb_ref[...], preferred_element_type=jnp.float32) + chb_ref[...]
    W = (1.0 - jax.nn.sigmoid(Pz)) * jnp.tanh(Ph)
    H = jnp.dot(W, wsum_ref[...], preferred_element_type=jnp.float32)
    out_ref[...] = (jnp.dot(jax.nn.relu(H), wo_ref[...],
                            preferred_element_type=jnp.float32) + bo_ref[...])


_dense = pl.pallas_call(
    _dense_body,
    grid=(NBLK,),
    in_specs=[
        pl.BlockSpec((BLK, 1), lambda i: (i, 0)),
        pl.BlockSpec((BLK, CW), lambda i: (i, 0)),
        pl.BlockSpec((BLK, CW), lambda i: (i, 0)),
        pl.BlockSpec((BLK, CW), lambda i: (i, 0)),
        pl.BlockSpec((BLK, CW), lambda i: (i, 0)),
        pl.BlockSpec((BLK, CW), lambda i: (i, 0)),
        pl.BlockSpec((BLK, CW), lambda i: (i, 0)),
        pl.BlockSpec((BLK, CW), lambda i: (i, 0)),
        pl.BlockSpec((BLK, CW), lambda i: (i, 0)),
        pl.BlockSpec((BLK, CW), lambda i: (i, 0)),
        pl.BlockSpec((FW, PERIODS * OUT), lambda i: (0, 0)),
        pl.BlockSpec((FW, PERIODS * OUT), lambda i: (0, 0)),
        pl.BlockSpec((1, PERIODS * OUT), lambda i: (0, 0)),
        pl.BlockSpec((1, PERIODS * OUT), lambda i: (0, 0)),
        pl.BlockSpec((PERIODS * OUT, OUT), lambda i: (0, 0)),
        pl.BlockSpec((OUT, PERIODS), lambda i: (0, 0)),
        pl.BlockSpec((1, PERIODS), lambda i: (0, 0)),
    ],
    out_specs=pl.BlockSpec((BLK, PERIODS), lambda i: (i, 0)),
    out_shape=jax.ShapeDtypeStruct((N, PERIODS), jnp.float32),
)


def kernel(x, edge_index, Wz, bz, Lz, lbz, Wr, br, Lr, lbr, Wh, bh, Lh, lbh,
           att, Wo, bo):
    del Wr, br, Lr, lbr

    src = edge_index[0].astype(jnp.int32)
    dst = edge_index[1].astype(jnp.int32)
    pad = (N + jnp.arange(EPAD - E, dtype=jnp.int32) % (NPAD - N))
    src_p = jnp.concatenate([src, pad]).reshape(EPAD // GROUP, GROUP)
    dst_p = jnp.concatenate([dst, pad]).reshape(EPAD // GROUP, GROUP)

    ones_g = jnp.ones((GROUP, DW), jnp.float32)
    zeros1 = jnp.zeros((NPAD, DW), jnp.float32)
    zeros_c = jnp.zeros((RPT, CW), jnp.float32)

    deg2 = _deg_kernel(dst_p, ones_g, zeros1)
    dinv, xs0, xs1, xs2 = _prep(deg2, x)
    yps = _msg_kernel(src_p, dst_p, xs0, xs1, xs2, zeros_c)

    probs = jax.nn.softmax(att)
    Az = Wz @ Lz[:OUT]
    cz = bz @ Lz[:OUT] + lbz
    Ah = Wh @ Lh[:OUT]
    ch = bh @ Lh[:OUT] + lbh
    eyeP = jnp.eye(PERIODS, dtype=jnp.float32)
    r = jnp.arange(FW)
    rows_map = (r % PERIODS) * F_IN + (r // PERIODS)
    Azb = jnp.kron(eyeP, Az)[rows_map]
    Ahb = jnp.kron(eyeP, Ah)[rows_map]
    czb = jnp.tile(cz, PERIODS)[None, :]
    chb = jnp.tile(ch, PERIODS)[None, :]
    Wsum = jnp.kron(probs[:, None], jnp.eye(OUT, dtype=jnp.float32))

    return _dense(dinv, *yps, xs0, xs1, xs2, Azb, Ahb, czb, chb, Wsum, Wo,
                  bo[None, :])


# speedup vs baseline: 1.1461x; 1.1461x over previous
import functools

import jax
import jax.numpy as jnp
from jax import lax
from jax.experimental import pallas as pl
from jax.experimental.pallas import tpu as pltpu
from jax.experimental.pallas import tpu_sc as plsc

N = 50000
E = 800000
F_IN = 8
OUT = 32
PERIODS = 12
FW = F_IN * PERIODS

NCORE = 2
NSUB = 16
NPAD = 50176
RPT = NPAD // NSUB

GROUP = 128
G0 = 120
G1 = 288
NGRP = NSUB * (G0 + G1)
EPAD = NGRP * GROUP

NCHUNK = 3
CW = 32

BLK = 512
NBLK = NPAD // BLK

_mesh = plsc.VectorSubcoreMesh(core_axis_name="c", subcore_axis_name="s")


DW = 16
DGA = 208
DGB = NGRP // NSUB - DGA


@functools.partial(
    pl.kernel,
    out_type=jax.ShapeDtypeStruct((NCORE, NPAD, DW), jnp.float32),
    mesh=_mesh,
    compiler_params=pltpu.CompilerParams(use_tc_tiling_on_sc=False),
    scratch_types=[
        pltpu.VMEM((DGA, GROUP), jnp.int32),
        pltpu.VMEM((GROUP, DW), jnp.float32),
        pltpu.VMEM_SHARED((NPAD, DW), jnp.float32),
    ],
)
def _deg_kernel(dst_hbm, ones_hbm, zeros1_hbm, deg_out, didx, ones_v, deg_sh):
    c = lax.axis_index("c")
    s = lax.axis_index("s")
    rbase = pl.multiple_of(s * RPT, 8)
    pltpu.sync_copy(zeros1_hbm.at[pl.ds(rbase, RPT)], deg_sh.at[pl.ds(rbase, RPT)])
    pltpu.sync_copy(ones_hbm, ones_v)

    def run_core(ngr, gbase):
        pltpu.sync_copy(dst_hbm.at[pl.ds(gbase, ngr)], didx.at[pl.ds(0, ngr)])
        plsc.subcore_barrier()

        @pl.loop(0, ngr)
        def _scatter(g):
            pltpu.sync_copy(ones_v, deg_sh.at[didx.at[g]], add=True)

    @pl.when(c == 0)
    def _():
        run_core(DGA, pl.multiple_of(s * DGA, 8))

    @pl.when(c == 1)
    def _():
        run_core(DGB, pl.multiple_of(NSUB * DGA + s * DGB, 8))

    plsc.subcore_barrier()
    pltpu.sync_copy(deg_sh.at[pl.ds(rbase, RPT)], deg_out.at[c, pl.ds(rbase, RPT)])


def _prep_body(deg_ref, x_ref, dinv_ref, xs0_ref, xs1_ref, xs2_ref):
    d = deg_ref[0, :, 0:1] + deg_ref[1, :, 0:1] + 1.0
    dv = lax.rsqrt(d)
    dinv_ref[...] = dv
    xs0_ref[...] = dv * x_ref[:, 0 * CW:1 * CW]
    xs1_ref[...] = dv * x_ref[:, 1 * CW:2 * CW]
    xs2_ref[...] = dv * x_ref[:, 2 * CW:3 * CW]


_prep = pl.pallas_call(
    _prep_body,
    grid=(NBLK,),
    in_specs=[
        pl.BlockSpec((NCORE, BLK, DW), lambda i: (0, i, 0)),
        pl.BlockSpec((BLK, FW), lambda i: (i, 0)),
    ],
    out_specs=[
        pl.BlockSpec((BLK, 1), lambda i: (i, 0)),
        pl.BlockSpec((BLK, CW), lambda i: (i, 0)),
        pl.BlockSpec((BLK, CW), lambda i: (i, 0)),
        pl.BlockSpec((BLK, CW), lambda i: (i, 0)),
    ],
    out_shape=[
        jax.ShapeDtypeStruct((NPAD, 1), jnp.float32),
        jax.ShapeDtypeStruct((NPAD, CW), jnp.float32),
        jax.ShapeDtypeStruct((NPAD, CW), jnp.float32),
        jax.ShapeDtypeStruct((NPAD, CW), jnp.float32),
    ],
)


IB = 24
NIB0 = G0 // IB
NIB1 = G1 // IB


@functools.partial(
    pl.kernel,
    out_type=[jax.ShapeDtypeStruct((NPAD, CW), jnp.float32)] * (NCORE * NCHUNK),
    mesh=_mesh,
    compiler_params=pltpu.CompilerParams(use_tc_tiling_on_sc=False),
    scratch_types=[
        pltpu.VMEM((IB, GROUP), jnp.int32),
        pltpu.VMEM((IB, GROUP), jnp.int32),
        pltpu.VMEM((GROUP, CW), jnp.float32),
        pltpu.VMEM((GROUP, CW), jnp.float32),
        pltpu.VMEM((GROUP, CW), jnp.float32),
        pltpu.VMEM((GROUP, CW), jnp.float32),
        pltpu.VMEM_SHARED((NPAD, CW), jnp.float32),
        pltpu.SemaphoreType.DMA,
        pltpu.SemaphoreType.DMA,
        pltpu.SemaphoreType.DMA,
        pltpu.SemaphoreType.DMA,
    ],
)
def _msg_kernel(src_hbm, dst_hbm, xs0_hbm, xs1_hbm, xs2_hbm, zeros_hbm,
                y00, y01, y02, y10, y11, y12,
                sidx, didx, rows0, rows1, rows2, rows3, y_sh,
                sem0, sem1, sem2, sem3):
    c = lax.axis_index("c")
    s = lax.axis_index("s")
    rows = (rows0, rows1, rows2, rows3)
    sems = (sem0, sem1, sem2, sem3)
    rbase = pl.multiple_of(s * RPT, 8)

    def run_core(nib, gbase, outs):
        for chunk, xs_hbm in enumerate((xs0_hbm, xs1_hbm, xs2_hbm)):
            pltpu.sync_copy(zeros_hbm, y_sh.at[pl.ds(rbase, RPT)])
            plsc.subcore_barrier()

            @pl.loop(0, nib)
            def _blocks(blk):
                bbase = pl.multiple_of(gbase + blk * IB, 8)
                pltpu.sync_copy(src_hbm.at[pl.ds(bbase, IB)], sidx)
                pltpu.sync_copy(dst_hbm.at[pl.ds(bbase, IB)], didx)

                for b in range(4):
                    pltpu.async_copy(xs_hbm.at[sidx.at[b]], rows[b], sems[b])

                @pl.loop(0, IB - 4, step=4)
                def _groups(g0):
                    for b in range(4):
                        g = g0 + b
                        pltpu.make_async_copy(xs_hbm.at[sidx.at[g]], rows[b],
                                              sems[b]).wait()
                        pltpu.sync_copy(rows[b], y_sh.at[didx.at[g]], add=True)
                        pltpu.async_copy(xs_hbm.at[sidx.at[g + 4]], rows[b],
                                         sems[b])

                for b in range(4):
                    g = IB - 4 + b
                    pltpu.make_async_copy(xs_hbm.at[sidx.at[g]], rows[b],
                                          sems[b]).wait()
                    pltpu.sync_copy(rows[b], y_sh.at[didx.at[g]], add=True)

            plsc.subcore_barrier()
            pltpu.sync_copy(y_sh.at[pl.ds(rbase, RPT)],
                            outs[chunk].at[pl.ds(rbase, RPT)])

    @pl.when(c == 0)
    def _():
        run_core(NIB0, pl.multiple_of(s * G0, 8), (y00, y01, y02))

    @pl.when(c == 1)
    def _():
        run_core(NIB1, pl.multiple_of(NSUB * G0 + s * G1, 8), (y10, y11, y12))


def _dense_body(dinv_ref, y00, y01, y02, y10, y11, y12,
                xs0_ref, xs1_ref, xs2_ref,
                azb_ref, ahb_ref, czb_ref, chb_ref, wsum_ref, wo_ref, bo_ref,
                out_ref):
    dv = dinv_ref[...]
    xs = (xs0_ref, xs1_ref, xs2_ref)
    yp0 = (y00, y01, y02)
    yp1 = (y10, y11, y12)
    y = [dv * (yp0[cc][...] + yp1[cc][...] + xs[cc][...]) for cc in range(NCHUNK)]
    Y = jnp.concatenate(y, axis=1)
    Pz = jnp.dot(Y, azb_ref[...], preferred_element_type=jnp.float32) + czb_ref[...]
    Ph = jnp.dot(Y, ahb_ref[...], preferred_element_type=jnp.float32) + chb_ref[...]
    W = (1.0 - jax.nn.sigmoid(Pz)) * jnp.tanh(Ph)
    H = jnp.dot(W, wsum_ref[...], preferred_element_type=jnp.float32)
    out_ref[...] = (jnp.dot(jax.nn.relu(H), wo_ref[...],
                            preferred_element_type=jnp.float32) + bo_ref[...])


_dense = pl.pallas_call(
    _dense_body,
    grid=(NBLK,),
    in_specs=[
        pl.BlockSpec((BLK, 1), lambda i: (i, 0)),
        pl.BlockSpec((BLK, CW), lambda i: (i, 0)),
        pl.BlockSpec((BLK, CW), lambda i: (i, 0)),
        pl.BlockSpec((BLK, CW), lambda i: (i, 0)),
        pl.BlockSpec((BLK, CW), lambda i: (i, 0)),
        pl.BlockSpec((BLK, CW), lambda i: (i, 0)),
        pl.BlockSpec((BLK, CW), lambda i: (i, 0)),
        pl.BlockSpec((BLK, CW), lambda i: (i, 0)),
        pl.BlockSpec((BLK, CW), lambda i: (i, 0)),
        pl.BlockSpec((BLK, CW), lambda i: (i, 0)),
        pl.BlockSpec((FW, PERIODS * OUT), lambda i: (0, 0)),
        pl.BlockSpec((FW, PERIODS * OUT), lambda i: (0, 0)),
        pl.BlockSpec((1, PERIODS * OUT), lambda i: (0, 0)),
        pl.BlockSpec((1, PERIODS * OUT), lambda i: (0, 0)),
        pl.BlockSpec((PERIODS * OUT, OUT), lambda i: (0, 0)),
        pl.BlockSpec((OUT, PERIODS), lambda i: (0, 0)),
        pl.BlockSpec((1, PERIODS), lambda i: (0, 0)),
    ],
    out_specs=pl.BlockSpec((BLK, PERIODS), lambda i: (i, 0)),
    out_shape=jax.ShapeDtypeStruct((N, PERIODS), jnp.float32),
)


def kernel(x, edge_index, Wz, bz, Lz, lbz, Wr, br, Lr, lbr, Wh, bh, Lh, lbh,
           att, Wo, bo):
    del Wr, br, Lr, lbr

    src = edge_index[0].astype(jnp.int32)
    dst = edge_index[1].astype(jnp.int32)
    pad = (N + jnp.arange(EPAD - E, dtype=jnp.int32) % (NPAD - N))
    src_p = jnp.concatenate([src, pad]).reshape(EPAD // GROUP, GROUP)
    dst_p = jnp.concatenate([dst, pad]).reshape(EPAD // GROUP, GROUP)

    ones_g = jnp.ones((GROUP, DW), jnp.float32)
    zeros1 = jnp.zeros((NPAD, DW), jnp.float32)
    zeros_c = jnp.zeros((RPT, CW), jnp.float32)

    deg2 = _deg_kernel(dst_p, ones_g, zeros1)
    dinv, xs0, xs1, xs2 = _prep(deg2, x.reshape(N, FW))
    yps = _msg_kernel(src_p, dst_p, xs0, xs1, xs2, zeros_c)

    probs = jax.nn.softmax(att)
    Az = Wz @ Lz[:OUT]
    cz = bz @ Lz[:OUT] + lbz
    Ah = Wh @ Lh[:OUT]
    ch = bh @ Lh[:OUT] + lbh
    eyeP = jnp.eye(PERIODS, dtype=jnp.float32)
    r = jnp.arange(FW)
    rows_map = (r % PERIODS) * F_IN + (r // PERIODS)
    Azb = jnp.kron(eyeP, Az)[rows_map]
    Ahb = jnp.kron(eyeP, Ah)[rows_map]
    czb = jnp.tile(cz, PERIODS)[None, :]
    chb = jnp.tile(ch, PERIODS)[None, :]
    Wsum = jnp.kron(probs[:, None], jnp.eye(OUT, dtype=jnp.float32))

    return _dense(dinv, *yps, xs0, xs1, xs2, Azb, Ahb, czb, chb, Wsum, Wo,
                  bo[None, :])


# speedup vs baseline: 1.2379x; 1.0802x over previous
import functools

import jax
import jax.numpy as jnp
from jax import lax
from jax.experimental import pallas as pl
from jax.experimental.pallas import tpu as pltpu
from jax.experimental.pallas import tpu_sc as plsc

N = 50000
E = 800000
F_IN = 8
OUT = 32
PERIODS = 12
FW = F_IN * PERIODS

NCORE = 2
NSUB = 16
NPAD = 50176
RPT = NPAD // NSUB

GROUP = 128
G0 = 120
G1 = 288
NGRP = NSUB * (G0 + G1)
EPAD = NGRP * GROUP

NCHUNK = 3
CW = 32

BLK = 1024
NBLK = NPAD // BLK

_mesh = plsc.VectorSubcoreMesh(core_axis_name="c", subcore_axis_name="s")


DW = 16
DGA = 208
DGB = NGRP // NSUB - DGA


@functools.partial(
    pl.kernel,
    out_type=jax.ShapeDtypeStruct((NCORE, NPAD, DW), jnp.float32),
    mesh=_mesh,
    compiler_params=pltpu.CompilerParams(use_tc_tiling_on_sc=False),
    scratch_types=[
        pltpu.VMEM((DGA, GROUP), jnp.int32),
        pltpu.VMEM((GROUP, DW), jnp.float32),
        pltpu.VMEM_SHARED((NPAD, DW), jnp.float32),
    ],
)
def _deg_kernel(dst_hbm, ones_hbm, zeros1_hbm, deg_out, didx, ones_v, deg_sh):
    c = lax.axis_index("c")
    s = lax.axis_index("s")
    rbase = pl.multiple_of(s * RPT, 8)
    pltpu.sync_copy(zeros1_hbm.at[pl.ds(rbase, RPT)], deg_sh.at[pl.ds(rbase, RPT)])
    pltpu.sync_copy(ones_hbm, ones_v)

    def run_core(ngr, gbase):
        pltpu.sync_copy(dst_hbm.at[pl.ds(gbase, ngr)], didx.at[pl.ds(0, ngr)])
        plsc.subcore_barrier()

        @pl.loop(0, ngr)
        def _scatter(g):
            pltpu.sync_copy(ones_v, deg_sh.at[didx.at[g]], add=True)

    @pl.when(c == 0)
    def _():
        run_core(DGA, pl.multiple_of(s * DGA, 8))

    @pl.when(c == 1)
    def _():
        run_core(DGB, pl.multiple_of(NSUB * DGA + s * DGB, 8))

    plsc.subcore_barrier()
    pltpu.sync_copy(deg_sh.at[pl.ds(rbase, RPT)], deg_out.at[c, pl.ds(rbase, RPT)])


def _prep_body(deg_ref, x_ref, dinv_ref, xs0_ref, xs1_ref, xs2_ref):
    d = deg_ref[0, :, 0:1] + deg_ref[1, :, 0:1] + 1.0
    dv = lax.rsqrt(d)
    dinv_ref[...] = dv
    xs0_ref[...] = dv * x_ref[:, 0 * CW:1 * CW]
    xs1_ref[...] = dv * x_ref[:, 1 * CW:2 * CW]
    xs2_ref[...] = dv * x_ref[:, 2 * CW:3 * CW]


_prep = pl.pallas_call(
    _prep_body,
    grid=(NBLK,),
    in_specs=[
        pl.BlockSpec((NCORE, BLK, DW), lambda i: (0, i, 0)),
        pl.BlockSpec((BLK, FW), lambda i: (i, 0)),
    ],
    out_specs=[
        pl.BlockSpec((BLK, 1), lambda i: (i, 0)),
        pl.BlockSpec((BLK, CW), lambda i: (i, 0)),
        pl.BlockSpec((BLK, CW), lambda i: (i, 0)),
        pl.BlockSpec((BLK, CW), lambda i: (i, 0)),
    ],
    out_shape=[
        jax.ShapeDtypeStruct((NPAD, 1), jnp.float32),
        jax.ShapeDtypeStruct((NPAD, CW), jnp.float32),
        jax.ShapeDtypeStruct((NPAD, CW), jnp.float32),
        jax.ShapeDtypeStruct((NPAD, CW), jnp.float32),
    ],
)


IB = 24
NIB0 = G0 // IB
NIB1 = G1 // IB


@functools.partial(
    pl.kernel,
    out_type=[jax.ShapeDtypeStruct((NPAD, CW), jnp.float32)] * (NCORE * NCHUNK),
    mesh=_mesh,
    compiler_params=pltpu.CompilerParams(use_tc_tiling_on_sc=False),
    scratch_types=[
        pltpu.VMEM((IB, GROUP), jnp.int32),
        pltpu.VMEM((IB, GROUP), jnp.int32),
        pltpu.VMEM((GROUP, CW), jnp.float32),
        pltpu.VMEM((GROUP, CW), jnp.float32),
        pltpu.VMEM((GROUP, CW), jnp.float32),
        pltpu.VMEM((GROUP, CW), jnp.float32),
        pltpu.VMEM_SHARED((NPAD, CW), jnp.float32),
        pltpu.SemaphoreType.DMA,
        pltpu.SemaphoreType.DMA,
        pltpu.SemaphoreType.DMA,
        pltpu.SemaphoreType.DMA,
    ],
)
def _msg_kernel(src_hbm, dst_hbm, xs0_hbm, xs1_hbm, xs2_hbm, zeros_hbm,
                y00, y01, y02, y10, y11, y12,
                sidx, didx, rows0, rows1, rows2, rows3, y_sh,
                sem0, sem1, sem2, sem3):
    c = lax.axis_index("c")
    s = lax.axis_index("s")
    rows = (rows0, rows1, rows2, rows3)
    sems = (sem0, sem1, sem2, sem3)
    rbase = pl.multiple_of(s * RPT, 8)

    def run_core(nib, gbase, outs):
        for chunk, xs_hbm in enumerate((xs0_hbm, xs1_hbm, xs2_hbm)):
            pltpu.sync_copy(zeros_hbm, y_sh.at[pl.ds(rbase, RPT)])
            plsc.subcore_barrier()

            @pl.loop(0, nib)
            def _blocks(blk):
                bbase = pl.multiple_of(gbase + blk * IB, 8)
                pltpu.sync_copy(src_hbm.at[pl.ds(bbase, IB)], sidx)
                pltpu.sync_copy(dst_hbm.at[pl.ds(bbase, IB)], didx)

                for b in range(4):
                    pltpu.async_copy(xs_hbm.at[sidx.at[b]], rows[b], sems[b])

                @pl.loop(0, IB - 4, step=4)
                def _groups(g0):
                    for b in range(4):
                        g = g0 + b
                        pltpu.make_async_copy(xs_hbm.at[sidx.at[g]], rows[b],
                                              sems[b]).wait()
                        pltpu.sync_copy(rows[b], y_sh.at[didx.at[g]], add=True)
                        pltpu.async_copy(xs_hbm.at[sidx.at[g + 4]], rows[b],
                                         sems[b])

                for b in range(4):
                    g = IB - 4 + b
                    pltpu.make_async_copy(xs_hbm.at[sidx.at[g]], rows[b],
                                          sems[b]).wait()
                    pltpu.sync_copy(rows[b], y_sh.at[didx.at[g]], add=True)

            plsc.subcore_barrier()
            pltpu.sync_copy(y_sh.at[pl.ds(rbase, RPT)],
                            outs[chunk].at[pl.ds(rbase, RPT)])

    @pl.when(c == 0)
    def _():
        run_core(NIB0, pl.multiple_of(s * G0, 8), (y00, y01, y02))

    @pl.when(c == 1)
    def _():
        run_core(NIB1, pl.multiple_of(NSUB * G0 + s * G1, 8), (y10, y11, y12))


def _dense_body(dinv_ref, y00, y01, y02, y10, y11, y12,
                xs0_ref, xs1_ref, xs2_ref,
                azb_ref, ahb_ref, czb_ref, chb_ref, wsum_ref, wo_ref, bo_ref,
                out_ref):
    dv = dinv_ref[...]
    xs = (xs0_ref, xs1_ref, xs2_ref)
    yp0 = (y00, y01, y02)
    yp1 = (y10, y11, y12)
    y = [dv * (yp0[cc][...] + yp1[cc][...] + xs[cc][...]) for cc in range(NCHUNK)]
    Y = jnp.concatenate(y, axis=1)
    Pz = jnp.dot(Y, azb_ref[...], preferred_element_type=jnp.float32) + czb_ref[...]
    Ph = jnp.dot(Y, ahb_ref[...], preferred_element_type=jnp.float32) + chb_ref[...]
    W = (1.0 - jax.nn.sigmoid(Pz)) * jnp.tanh(Ph)
    H = jnp.dot(W, wsum_ref[...], preferred_element_type=jnp.float32)
    out_ref[...] = (jnp.dot(jax.nn.relu(H), wo_ref[...],
                            preferred_element_type=jnp.float32) + bo_ref[...])


_dense = pl.pallas_call(
    _dense_body,
    grid=(NBLK,),
    in_specs=[
        pl.BlockSpec((BLK, 1), lambda i: (i, 0)),
        pl.BlockSpec((BLK, CW), lambda i: (i, 0)),
        pl.BlockSpec((BLK, CW), lambda i: (i, 0)),
        pl.BlockSpec((BLK, CW), lambda i: (i, 0)),
        pl.BlockSpec((BLK, CW), lambda i: (i, 0)),
        pl.BlockSpec((BLK, CW), lambda i: (i, 0)),
        pl.BlockSpec((BLK, CW), lambda i: (i, 0)),
        pl.BlockSpec((BLK, CW), lambda i: (i, 0)),
        pl.BlockSpec((BLK, CW), lambda i: (i, 0)),
        pl.BlockSpec((BLK, CW), lambda i: (i, 0)),
        pl.BlockSpec((FW, PERIODS * OUT), lambda i: (0, 0)),
        pl.BlockSpec((FW, PERIODS * OUT), lambda i: (0, 0)),
        pl.BlockSpec((1, PERIODS * OUT), lambda i: (0, 0)),
        pl.BlockSpec((1, PERIODS * OUT), lambda i: (0, 0)),
        pl.BlockSpec((PERIODS * OUT, OUT), lambda i: (0, 0)),
        pl.BlockSpec((OUT, PERIODS), lambda i: (0, 0)),
        pl.BlockSpec((1, PERIODS), lambda i: (0, 0)),
    ],
    out_specs=pl.BlockSpec((BLK, PERIODS), lambda i: (i, 0)),
    out_shape=jax.ShapeDtypeStruct((N, PERIODS), jnp.float32),
)


def kernel(x, edge_index, Wz, bz, Lz, lbz, Wr, br, Lr, lbr, Wh, bh, Lh, lbh,
           att, Wo, bo):
    del Wr, br, Lr, lbr

    src = edge_index[0].astype(jnp.int32)
    dst = edge_index[1].astype(jnp.int32)
    pad = (N + jnp.arange(EPAD - E, dtype=jnp.int32) % (NPAD - N))
    src_p = jnp.concatenate([src, pad]).reshape(EPAD // GROUP, GROUP)
    dst_p = jnp.concatenate([dst, pad]).reshape(EPAD // GROUP, GROUP)

    ones_g = jnp.ones((GROUP, DW), jnp.float32)
    zeros1 = jnp.zeros((NPAD, DW), jnp.float32)
    zeros_c = jnp.zeros((RPT, CW), jnp.float32)

    deg2 = _deg_kernel(dst_p, ones_g, zeros1)
    dinv, xs0, xs1, xs2 = _prep(deg2, x.reshape(N, FW))
    yps = _msg_kernel(src_p, dst_p, xs0, xs1, xs2, zeros_c)

    probs = jax.nn.softmax(att)
    Az = Wz @ Lz[:OUT]
    cz = bz @ Lz[:OUT] + lbz
    Ah = Wh @ Lh[:OUT]
    ch = bh @ Lh[:OUT] + lbh
    eyeP = jnp.eye(PERIODS, dtype=jnp.float32)
    r = jnp.arange(FW)
    rows_map = (r % PERIODS) * F_IN + (r // PERIODS)
    Azb = jnp.kron(eyeP, Az)[rows_map]
    Ahb = jnp.kron(eyeP, Ah)[rows_map]
    czb = jnp.tile(cz, PERIODS)[None, :]
    chb = jnp.tile(ch, PERIODS)[None, :]
    Wsum = jnp.kron(probs[:, None], jnp.eye(OUT, dtype=jnp.float32))

    return _dense(dinv, *yps, xs0, xs1, xs2, Azb, Ahb, czb, chb, Wsum, Wo,
                  bo[None, :])


# speedup vs baseline: 1.2785x; 1.0328x over previous
import functools

import jax
import jax.numpy as jnp
from jax import lax
from jax.experimental import pallas as pl
from jax.experimental.pallas import tpu as pltpu
from jax.experimental.pallas import tpu_sc as plsc

N = 50000
E = 800000
F_IN = 8
OUT = 32
PERIODS = 12
FW = F_IN * PERIODS

NCORE = 2
NSUB = 16
NPAD = 50176
RPT = NPAD // NSUB

GROUP = 128
G0 = 120
G1 = 288
NGRP = NSUB * (G0 + G1)
EPAD = NGRP * GROUP

NCHUNK = 3
CW = 32

BLK = 3136
NBLK = NPAD // BLK

_mesh = plsc.VectorSubcoreMesh(core_axis_name="c", subcore_axis_name="s")


DW = 16
DGA = 208
DGB = NGRP // NSUB - DGA


@functools.partial(
    pl.kernel,
    out_type=jax.ShapeDtypeStruct((NCORE, NPAD, DW), jnp.float32),
    mesh=_mesh,
    compiler_params=pltpu.CompilerParams(use_tc_tiling_on_sc=False),
    scratch_types=[
        pltpu.VMEM((DGA, GROUP), jnp.int32),
        pltpu.VMEM((GROUP, DW), jnp.float32),
        pltpu.VMEM_SHARED((NPAD, DW), jnp.float32),
    ],
)
def _deg_kernel(dst_hbm, ones_hbm, zeros1_hbm, deg_out, didx, ones_v, deg_sh):
    c = lax.axis_index("c")
    s = lax.axis_index("s")
    rbase = pl.multiple_of(s * RPT, 8)
    pltpu.sync_copy(zeros1_hbm.at[pl.ds(rbase, RPT)], deg_sh.at[pl.ds(rbase, RPT)])
    pltpu.sync_copy(ones_hbm, ones_v)

    def run_core(ngr, gbase):
        pltpu.sync_copy(dst_hbm.at[pl.ds(gbase, ngr)], didx.at[pl.ds(0, ngr)])
        plsc.subcore_barrier()

        @pl.loop(0, ngr)
        def _scatter(g):
            pltpu.sync_copy(ones_v, deg_sh.at[didx.at[g]], add=True)

    @pl.when(c == 0)
    def _():
        run_core(DGA, pl.multiple_of(s * DGA, 8))

    @pl.when(c == 1)
    def _():
        run_core(DGB, pl.multiple_of(NSUB * DGA + s * DGB, 8))

    plsc.subcore_barrier()
    pltpu.sync_copy(deg_sh.at[pl.ds(rbase, RPT)], deg_out.at[c, pl.ds(rbase, RPT)])


def _prep_body(deg_ref, x_ref, dinv_ref, xs0_ref, xs1_ref, xs2_ref):
    d = deg_ref[0, :, 0:1] + deg_ref[1, :, 0:1] + 1.0
    dv = lax.rsqrt(d)
    dinv_ref[...] = dv
    xs0_ref[...] = dv * x_ref[:, 0 * CW:1 * CW]
    xs1_ref[...] = dv * x_ref[:, 1 * CW:2 * CW]
    xs2_ref[...] = dv * x_ref[:, 2 * CW:3 * CW]


_prep = pl.pallas_call(
    _prep_body,
    grid=(NBLK,),
    in_specs=[
        pl.BlockSpec((NCORE, BLK, DW), lambda i: (0, i, 0)),
        pl.BlockSpec((BLK, FW), lambda i: (i, 0)),
    ],
    out_specs=[
        pl.BlockSpec((BLK, 1), lambda i: (i, 0)),
        pl.BlockSpec((BLK, CW), lambda i: (i, 0)),
        pl.BlockSpec((BLK, CW), lambda i: (i, 0)),
        pl.BlockSpec((BLK, CW), lambda i: (i, 0)),
    ],
    out_shape=[
        jax.ShapeDtypeStruct((NPAD, 1), jnp.float32),
        jax.ShapeDtypeStruct((NPAD, CW), jnp.float32),
        jax.ShapeDtypeStruct((NPAD, CW), jnp.float32),
        jax.ShapeDtypeStruct((NPAD, CW), jnp.float32),
    ],
)


IB = 24
NIB0 = G0 // IB
NIB1 = G1 // IB


@functools.partial(
    pl.kernel,
    out_type=[jax.ShapeDtypeStruct((NPAD, CW), jnp.float32)] * (NCORE * NCHUNK),
    mesh=_mesh,
    compiler_params=pltpu.CompilerParams(use_tc_tiling_on_sc=False),
    scratch_types=[
        pltpu.VMEM((IB, GROUP), jnp.int32),
        pltpu.VMEM((IB, GROUP), jnp.int32),
        pltpu.VMEM((GROUP, CW), jnp.float32),
        pltpu.VMEM((GROUP, CW), jnp.float32),
        pltpu.VMEM((GROUP, CW), jnp.float32),
        pltpu.VMEM((GROUP, CW), jnp.float32),
        pltpu.VMEM_SHARED((NPAD, CW), jnp.float32),
        pltpu.SemaphoreType.DMA,
        pltpu.SemaphoreType.DMA,
        pltpu.SemaphoreType.DMA,
        pltpu.SemaphoreType.DMA,
    ],
)
def _msg_kernel(src_hbm, dst_hbm, xs0_hbm, xs1_hbm, xs2_hbm, zeros_hbm,
                y00, y01, y02, y10, y11, y12,
                sidx, didx, rows0, rows1, rows2, rows3, y_sh,
                sem0, sem1, sem2, sem3):
    c = lax.axis_index("c")
    s = lax.axis_index("s")
    rows = (rows0, rows1, rows2, rows3)
    sems = (sem0, sem1, sem2, sem3)
    rbase = pl.multiple_of(s * RPT, 8)

    def run_core(nib, gbase, outs):
        for chunk, xs_hbm in enumerate((xs0_hbm, xs1_hbm, xs2_hbm)):
            pltpu.sync_copy(zeros_hbm, y_sh.at[pl.ds(rbase, RPT)])
            plsc.subcore_barrier()

            @pl.loop(0, nib)
            def _blocks(blk):
                bbase = pl.multiple_of(gbase + blk * IB, 8)
                pltpu.sync_copy(src_hbm.at[pl.ds(bbase, IB)], sidx)
                pltpu.sync_copy(dst_hbm.at[pl.ds(bbase, IB)], didx)

                for b in range(4):
                    pltpu.async_copy(xs_hbm.at[sidx.at[b]], rows[b], sems[b])

                @pl.loop(0, IB - 4, step=4)
                def _groups(g0):
                    for b in range(4):
                        g = g0 + b
                        pltpu.make_async_copy(xs_hbm.at[sidx.at[g]], rows[b],
                                              sems[b]).wait()
                        pltpu.sync_copy(rows[b], y_sh.at[didx.at[g]], add=True)
                        pltpu.async_copy(xs_hbm.at[sidx.at[g + 4]], rows[b],
                                         sems[b])

                for b in range(4):
                    g = IB - 4 + b
                    pltpu.make_async_copy(xs_hbm.at[sidx.at[g]], rows[b],
                                          sems[b]).wait()
                    pltpu.sync_copy(rows[b], y_sh.at[didx.at[g]], add=True)

            plsc.subcore_barrier()
            pltpu.sync_copy(y_sh.at[pl.ds(rbase, RPT)],
                            outs[chunk].at[pl.ds(rbase, RPT)])

    @pl.when(c == 0)
    def _():
        run_core(NIB0, pl.multiple_of(s * G0, 8), (y00, y01, y02))

    @pl.when(c == 1)
    def _():
        run_core(NIB1, pl.multiple_of(NSUB * G0 + s * G1, 8), (y10, y11, y12))


def _dense_body(dinv_ref, y00, y01, y02, y10, y11, y12,
                xs0_ref, xs1_ref, xs2_ref,
                azb_ref, ahb_ref, czb_ref, chb_ref, wsum_ref, wo_ref, bo_ref,
                out_ref):
    dv = dinv_ref[...]
    xs = (xs0_ref, xs1_ref, xs2_ref)
    yp0 = (y00, y01, y02)
    yp1 = (y10, y11, y12)
    y = [dv * (yp0[cc][...] + yp1[cc][...] + xs[cc][...]) for cc in range(NCHUNK)]
    Y = jnp.concatenate(y, axis=1)
    Pz = jnp.dot(Y, azb_ref[...], preferred_element_type=jnp.float32) + czb_ref[...]
    Ph = jnp.dot(Y, ahb_ref[...], preferred_element_type=jnp.float32) + chb_ref[...]
    W = (1.0 - jax.nn.sigmoid(Pz)) * jnp.tanh(Ph)
    H = jnp.dot(W, wsum_ref[...], preferred_element_type=jnp.float32)
    out_ref[...] = (jnp.dot(jax.nn.relu(H), wo_ref[...],
                            preferred_element_type=jnp.float32) + bo_ref[...])


_dense = pl.pallas_call(
    _dense_body,
    grid=(NBLK,),
    in_specs=[
        pl.BlockSpec((BLK, 1), lambda i: (i, 0)),
        pl.BlockSpec((BLK, CW), lambda i: (i, 0)),
        pl.BlockSpec((BLK, CW), lambda i: (i, 0)),
        pl.BlockSpec((BLK, CW), lambda i: (i, 0)),
        pl.BlockSpec((BLK, CW), lambda i: (i, 0)),
        pl.BlockSpec((BLK, CW), lambda i: (i, 0)),
        pl.BlockSpec((BLK, CW), lambda i: (i, 0)),
        pl.BlockSpec((BLK, CW), lambda i: (i, 0)),
        pl.BlockSpec((BLK, CW), lambda i: (i, 0)),
        pl.BlockSpec((BLK, CW), lambda i: (i, 0)),
        pl.BlockSpec((FW, PERIODS * OUT), lambda i: (0, 0)),
        pl.BlockSpec((FW, PERIODS * OUT), lambda i: (0, 0)),
        pl.BlockSpec((1, PERIODS * OUT), lambda i: (0, 0)),
        pl.BlockSpec((1, PERIODS * OUT), lambda i: (0, 0)),
        pl.BlockSpec((PERIODS * OUT, OUT), lambda i: (0, 0)),
        pl.BlockSpec((OUT, PERIODS), lambda i: (0, 0)),
        pl.BlockSpec((1, PERIODS), lambda i: (0, 0)),
    ],
    out_specs=pl.BlockSpec((BLK, PERIODS), lambda i: (i, 0)),
    out_shape=jax.ShapeDtypeStruct((N, PERIODS), jnp.float32),
)


def kernel(x, edge_index, Wz, bz, Lz, lbz, Wr, br, Lr, lbr, Wh, bh, Lh, lbh,
           att, Wo, bo):
    del Wr, br, Lr, lbr

    src = edge_index[0].astype(jnp.int32)
    dst = edge_index[1].astype(jnp.int32)
    pad = (N + jnp.arange(EPAD - E, dtype=jnp.int32) % (NPAD - N))
    src_p = jnp.concatenate([src, pad]).reshape(EPAD // GROUP, GROUP)
    dst_p = jnp.concatenate([dst, pad]).reshape(EPAD // GROUP, GROUP)

    ones_g = jnp.ones((GROUP, DW), jnp.float32)
    zeros1 = jnp.zeros((NPAD, DW), jnp.float32)
    zeros_c = jnp.zeros((RPT, CW), jnp.float32)

    deg2 = _deg_kernel(dst_p, ones_g, zeros1)
    dinv, xs0, xs1, xs2 = _prep(deg2, x.reshape(N, FW))
    yps = _msg_kernel(src_p, dst_p, xs0, xs1, xs2, zeros_c)

    probs = jax.nn.softmax(att)
    Az = Wz @ Lz[:OUT]
    cz = bz @ Lz[:OUT] + lbz
    Ah = Wh @ Lh[:OUT]
    ch = bh @ Lh[:OUT] + lbh
    eyeP = jnp.eye(PERIODS, dtype=jnp.float32)
    r = jnp.arange(FW)
    rows_map = (r % PERIODS) * F_IN + (r // PERIODS)
    Azb = jnp.kron(eyeP, Az)[rows_map]
    Ahb = jnp.kron(eyeP, Ah)[rows_map]
    czb = jnp.tile(cz, PERIODS)[None, :]
    chb = jnp.tile(ch, PERIODS)[None, :]
    Wsum = jnp.kron(probs[:, None], jnp.eye(OUT, dtype=jnp.float32))

    return _dense(dinv, *yps, xs0, xs1, xs2, Azb, Ahb, czb, chb, Wsum, Wo,
                  bo[None, :])
